# baseline (speedup 1.0000x reference)
import functools

import jax
import jax.numpy as jnp
from jax import lax
from jax.experimental import pallas as pl
from jax.experimental.pallas import tpu as pltpu
from jax.experimental.pallas import tpu_sc as plsc

N = 10000
D = 128
NP = 10240
NW = 32
CHUNK = 128
RPT = NP // 16

_mesh = plsc.VectorSubcoreMesh(core_axis_name="c", subcore_axis_name="s")


def _deg_body(idx_hbm, zer_hbm, out_hbm, idx_v, ones_v, dacc, sem):
    nchp = idx_hbm.shape[1]
    c = lax.axis_index("c")
    s = lax.axis_index("s")
    wid = c * 16 + s

    @pl.loop(0, CHUNK // 16)
    def _(i):
        ones_v[pl.ds(i * 16, 16)] = jnp.ones((16,), jnp.float32)

    pltpu.sync_copy(zer_hbm, dacc.at[pl.ds(s * RPT, RPT)])
    pltpu.sync_copy(idx_hbm.at[wid], idx_v)
    plsc.subcore_barrier()

    @pl.loop(0, nchp)
    def _(j):
        pltpu.async_copy(ones_v, dacc.at[idx_v.at[j, 1]], sem, add=True)

    @pl.loop(0, nchp)
    def _(j):
        pltpu.make_async_copy(ones_v, dacc.at[idx_v.at[0, 1]], sem).wait()

    plsc.subcore_barrier()
    sl = pl.ds(s * RPT, RPT)
    pltpu.sync_copy(dacc.at[sl], out_hbm.at[c, sl])


def _deg_call(idxp, zer1):
    nchp = idxp.shape[1]
    f = functools.partial(
        pl.kernel,
        out_type=jax.ShapeDtypeStruct((2, NP), jnp.float32),
        mesh=_mesh,
        scratch_types=[
            pltpu.VMEM((nchp, 2, CHUNK), jnp.int32),
            pltpu.VMEM((CHUNK,), jnp.float32),
            pltpu.VMEM_SHARED((NP,), jnp.float32),
            pltpu.SemaphoreType.DMA,
        ],
    )(_deg_body)
    return f(idxp, zer1)


def _edge_body(g_hbm, idx_hbm, zer_hbm, out_hbm, ring, buf_a, buf_b, acc,
               sem_i, sem_a, sem_b, sem_sa, sem_sb):
    nch = idx_hbm.shape[1] - 2
    c = lax.axis_index("c")
    s = lax.axis_index("s")
    wid = c * 16 + s

    def idx_fetch(slot, j):
        pltpu.async_copy(idx_hbm.at[wid, pl.ds(j, 2)], ring.at[slot], sem_i)

    def wait_idx():
        pltpu.make_async_copy(idx_hbm.at[wid, pl.ds(0, 2)], ring.at[0], sem_i).wait()

    def gather(slot, k, buf, sem):
        pltpu.async_copy(g_hbm.at[ring.at[slot, k, 0]], buf, sem)

    def wait_g(buf, sem):
        pltpu.make_async_copy(g_hbm.at[ring.at[0, 0, 0]], buf, sem).wait()

    def scat(slot, k, buf, sem):
        pltpu.async_copy(buf, acc.at[ring.at[slot, k, 1]], sem, add=True)

    def wait_s(buf, sem):
        pltpu.make_async_copy(buf, acc.at[ring.at[0, 0, 1]], sem).wait()

    pltpu.sync_copy(zer_hbm, acc.at[pl.ds(s * RPT, RPT)])
    idx_fetch(0, 0)
    wait_idx()
    idx_fetch(1, 2)
    gather(0, 0, buf_a, sem_a)
    gather(0, 1, buf_b, sem_b)
    plsc.subcore_barrier()

    @pl.loop(0, nch - 2, step=2)
    def _(j):
        p = (j // 2) % 2
        q = 1 - p
        wait_g(buf_a, sem_a)
        scat(p, 0, buf_a, sem_sa)
        wait_g(buf_b, sem_b)
        scat(p, 1, buf_b, sem_sb)
        wait_idx()
        wait_s(buf_a, sem_sa)
        gather(q, 0, buf_a, sem_a)
        wait_s(buf_b, sem_sb)
        gather(q, 1, buf_b, sem_b)
        idx_fetch(p, j + 4)

    pe = ((nch - 2) // 2) % 2
    wait_g(buf_a, sem_a)
    scat(pe, 0, buf_a, sem_sa)
    wait_g(buf_b, sem_b)
    scat(pe, 1, buf_b, sem_sb)
    wait_idx()
    wait_s(buf_a, sem_sa)
    wait_s(buf_b, sem_sb)

    plsc.subcore_barrier()
    sl = pl.ds(s * RPT, RPT)
    pltpu.sync_copy(acc.at[sl], out_hbm.at[c, sl])


def _edge_call(g, idxp, zer2):
    nchp = idxp.shape[1]
    f = functools.partial(
        pl.kernel,
        out_type=jax.ShapeDtypeStruct((2, NP, D), jnp.float32),
        mesh=_mesh,
        scratch_types=[
            pltpu.VMEM((2, 2, 2, CHUNK), jnp.int32),
            pltpu.VMEM((CHUNK, D), jnp.float32),
            pltpu.VMEM((CHUNK, D), jnp.float32),
            pltpu.VMEM_SHARED((NP, D), jnp.float32),
            pltpu.SemaphoreType.DMA,
            pltpu.SemaphoreType.DMA,
            pltpu.SemaphoreType.DMA,
            pltpu.SemaphoreType.DMA,
            pltpu.SemaphoreType.DMA,
        ],
    )(_edge_body)
    return f(g, idxp, zer2)


def _tc1_body(x_ref, w_ref, degp_ref, o_ref):
    dinv = lax.rsqrt(degp_ref[0] + degp_ref[1] + 1.0)
    h = jnp.dot(x_ref[...], w_ref[...], preferred_element_type=jnp.float32,
                precision=lax.Precision.HIGHEST)
    o_ref[...] = h * dinv


def _tc2_body(s_ref, g_ref, degp_ref, w_ref, b_ref, o_ref):
    dinv = lax.rsqrt(degp_ref[0] + degp_ref[1] + 1.0)
    u = jnp.maximum(dinv * (s_ref[0] + s_ref[1] + g_ref[...]) + b_ref[...], 0.0)
    h = jnp.dot(u, w_ref[...], preferred_element_type=jnp.float32,
                precision=lax.Precision.HIGHEST)
    o_ref[...] = h * dinv


def _tc3_body(s_ref, g_ref, degp_ref, b_ref, o_ref):
    dinv = lax.rsqrt(degp_ref[0] + degp_ref[1] + 1.0)
    o_ref[...] = dinv * (s_ref[0] + s_ref[1] + g_ref[...]) + b_ref[...]


_out_np = jax.ShapeDtypeStruct((NP, D), jnp.float32)
_tc1 = pl.pallas_call(_tc1_body, out_shape=_out_np)
_tc2 = pl.pallas_call(_tc2_body, out_shape=_out_np)
_tc3 = pl.pallas_call(_tc3_body, out_shape=_out_np)


def kernel(x, edge_index, cache_name, W1, b1, W2, b2):
    e = edge_index.shape[1]
    blk = NW * CHUNK * 2
    ep = ((e + blk - 1) // blk) * blk
    nch = ep // (NW * CHUNK)
    pad = ep - e
    padv = N + (jnp.arange(pad, dtype=jnp.int32) % 32)
    src = jnp.concatenate([edge_index[0], padv]).reshape(NW, nch, CHUNK)
    dst = jnp.concatenate([edge_index[1], padv]).reshape(NW, nch, CHUNK)
    idxp = jnp.pad(jnp.stack([src, dst], axis=2), ((0, 0), (0, 2), (0, 0), (0, 0)),
                   constant_values=N)
    xp = jnp.pad(x, ((0, NP - N), (0, 0)))
    zer1 = jnp.zeros((RPT,), jnp.float32)
    zer2 = jnp.zeros((RPT, D), jnp.float32)

    degp = _deg_call(idxp, zer1)[:, :, None]
    g1 = _tc1(xp, W1, degp)
    s1 = _edge_call(g1, idxp, zer2)
    g2 = _tc2(s1, g1, degp, W2, b1.reshape(1, D))
    s2 = _edge_call(g2, idxp, zer2)
    out = _tc3(s2, g2, degp, b2.reshape(1, D))
    return out[:N]


# speedup vs baseline: 1.2353x; 1.2353x over previous
import functools

import jax
import jax.numpy as jnp
from jax import lax
from jax.experimental import pallas as pl
from jax.experimental.pallas import tpu as pltpu
from jax.experimental.pallas import tpu_sc as plsc

N = 10000
D = 128
NP = 10240
NW = 32
CHUNK = 128
RPT = NP // 16

_mesh = plsc.VectorSubcoreMesh(core_axis_name="c", subcore_axis_name="s")


def _deg_body(idx_hbm, zer_hbm, out_hbm, idx_v, ones_v, dacc, sem):
    nchp = idx_hbm.shape[1]
    c = lax.axis_index("c")
    s = lax.axis_index("s")
    wid = c * 16 + s

    @pl.loop(0, CHUNK // 16)
    def _(i):
        ones_v[pl.ds(i * 16, 16)] = jnp.ones((16,), jnp.float32)

    pltpu.sync_copy(zer_hbm, dacc.at[pl.ds(s * RPT, RPT)])
    pltpu.sync_copy(idx_hbm.at[wid], idx_v)
    plsc.subcore_barrier()

    @pl.loop(0, nchp)
    def _(j):
        pltpu.async_copy(ones_v, dacc.at[idx_v.at[j, 1]], sem, add=True)

    @pl.loop(0, nchp)
    def _(j):
        pltpu.make_async_copy(ones_v, dacc.at[idx_v.at[0, 1]], sem).wait()

    plsc.subcore_barrier()
    sl = pl.ds(s * RPT, RPT)
    pltpu.sync_copy(dacc.at[sl], out_hbm.at[c, sl])


def _deg_call(idxp, zer1):
    nchp = idxp.shape[1]
    f = functools.partial(
        pl.kernel,
        out_type=jax.ShapeDtypeStruct((2, NP), jnp.float32),
        mesh=_mesh,
        scratch_types=[
            pltpu.VMEM((nchp, 2, CHUNK), jnp.int32),
            pltpu.VMEM((CHUNK,), jnp.float32),
            pltpu.VMEM_SHARED((NP,), jnp.float32),
            pltpu.SemaphoreType.DMA,
        ],
    )(_deg_body)
    return f(idxp, zer1)


def _edge_body(g_hbm, idx_hbm, zer_hbm, out_hbm, ring, buf_a, buf_b, acc,
               sem_i, sem_a, sem_b):
    nch = idx_hbm.shape[1] - 2
    c = lax.axis_index("c")
    s = lax.axis_index("s")
    wid = c * 16 + s

    def idx_fetch(slot, j):
        pltpu.async_copy(idx_hbm.at[wid, pl.ds(j, 2)], ring.at[slot], sem_i)

    def wait_idx():
        pltpu.make_async_copy(idx_hbm.at[wid, pl.ds(0, 2)], ring.at[0], sem_i).wait()

    def gather(slot, k, buf, sem):
        pltpu.async_copy(g_hbm.at[ring.at[slot, k, 0]], buf, sem)

    def wait_g(buf, sem):
        pltpu.make_async_copy(g_hbm.at[ring.at[0, 0, 0]], buf, sem).wait()

    def scat(slot, k, buf):
        pltpu.sync_copy(buf, acc.at[ring.at[slot, k, 1]], add=True)

    idx_fetch(0, 0)
    pltpu.sync_copy(zer_hbm, acc.at[pl.ds(s * RPT, RPT)])
    wait_idx()
    idx_fetch(1, 2)
    gather(0, 0, buf_a, sem_a)
    gather(0, 1, buf_b, sem_b)
    plsc.subcore_barrier()

    @pl.loop(0, nch - 2, step=2)
    def _(j):
        p = (j // 2) % 2
        q = 1 - p
        wait_g(buf_a, sem_a)
        scat(p, 0, buf_a)
        wait_idx()
        gather(q, 0, buf_a, sem_a)
        wait_g(buf_b, sem_b)
        scat(p, 1, buf_b)
        gather(q, 1, buf_b, sem_b)
        idx_fetch(p, j + 4)

    pe = ((nch - 2) // 2) % 2
    wait_g(buf_a, sem_a)
    scat(pe, 0, buf_a)
    wait_g(buf_b, sem_b)
    scat(pe, 1, buf_b)
    wait_idx()

    plsc.subcore_barrier()
    sl = pl.ds(s * RPT, RPT)
    pltpu.sync_copy(acc.at[sl], out_hbm.at[c, sl])


def _edge_call(g, idxp, zer2):
    nchp = idxp.shape[1]
    f = functools.partial(
        pl.kernel,
        out_type=jax.ShapeDtypeStruct((2, NP, D), jnp.float32),
        mesh=_mesh,
        scratch_types=[
            pltpu.VMEM((2, 2, 2, CHUNK), jnp.int32),
            pltpu.VMEM((CHUNK, D), jnp.float32),
            pltpu.VMEM((CHUNK, D), jnp.float32),
            pltpu.VMEM_SHARED((NP, D), jnp.float32),
            pltpu.SemaphoreType.DMA,
            pltpu.SemaphoreType.DMA,
            pltpu.SemaphoreType.DMA,
        ],
    )(_edge_body)
    return f(g, idxp, zer2)


def _tc1_body(x_ref, w_ref, degp_ref, o_ref):
    dinv = lax.rsqrt(degp_ref[0] + degp_ref[1] + 1.0)
    h = jnp.dot(x_ref[...], w_ref[...], preferred_element_type=jnp.float32,
                precision=lax.Precision.HIGHEST)
    o_ref[...] = h * dinv


def _tc2_body(s_ref, g_ref, degp_ref, w_ref, b_ref, o_ref):
    dinv = lax.rsqrt(degp_ref[0] + degp_ref[1] + 1.0)
    u = jnp.maximum(dinv * (s_ref[0] + s_ref[1] + g_ref[...]) + b_ref[...], 0.0)
    h = jnp.dot(u, w_ref[...], preferred_element_type=jnp.float32,
                precision=lax.Precision.HIGHEST)
    o_ref[...] = h * dinv


def _tc3_body(s_ref, g_ref, degp_ref, b_ref, o_ref):
    dinv = lax.rsqrt(degp_ref[0] + degp_ref[1] + 1.0)
    o_ref[...] = dinv * (s_ref[0] + s_ref[1] + g_ref[...]) + b_ref[...]


_out_np = jax.ShapeDtypeStruct((NP, D), jnp.float32)
_tc1 = pl.pallas_call(_tc1_body, out_shape=_out_np)
_tc2 = pl.pallas_call(_tc2_body, out_shape=_out_np)
_tc3 = pl.pallas_call(_tc3_body, out_shape=_out_np)


def kernel(x, edge_index, cache_name, W1, b1, W2, b2):
    e = edge_index.shape[1]
    blk = NW * CHUNK * 2
    ep = ((e + blk - 1) // blk) * blk
    nch = ep // (NW * CHUNK)
    pad = ep - e
    padv = N + (jnp.arange(pad, dtype=jnp.int32) % 32)
    src = jnp.concatenate([edge_index[0], padv]).reshape(NW, nch, CHUNK)
    dst = jnp.concatenate([edge_index[1], padv]).reshape(NW, nch, CHUNK)
    idxp = jnp.pad(jnp.stack([src, dst], axis=2), ((0, 0), (0, 2), (0, 0), (0, 0)),
                   constant_values=N)
    xp = jnp.pad(x, ((0, NP - N), (0, 0)))
    zer1 = jnp.zeros((RPT,), jnp.float32)
    zer2 = jnp.zeros((RPT, D), jnp.float32)

    degp = _deg_call(idxp, zer1)[:, :, None]
    g1 = _tc1(xp, W1, degp)
    s1 = _edge_call(g1, idxp, zer2)
    g2 = _tc2(s1, g1, degp, W2, b1.reshape(1, D))
    s2 = _edge_call(g2, idxp, zer2)
    out = _tc3(s2, g2, degp, b2.reshape(1, D))
    return out[:N]
